# baseline (speedup 1.0000x reference)
import functools

import jax
import jax.numpy as jnp
from jax import lax
from jax.experimental import pallas as pl
from jax.experimental.pallas import tpu as pltpu
from jax.experimental.pallas import tpu_sc as plsc

B = 4096
L = 50
TEXT_DIM = 64
N_CAT = 26
CAT_VOCAB = 100000
TEXT_VOCAB = 100000
CAT_DIM = 32
N_NUM = 13
NUM_HID = 64
HIDDEN = 256
NUM_CLASSES = 10

NC = 2
NS = 16
NW = NC * NS
COLS_PER_W = TEXT_DIM // NW
LANES = 16
NV = B // LANES


def _sc_features(emb_title_t, emb_desc_t, cat_t, title_idx_t, desc_idx_t,
                 cat_idx_t):
  mesh = plsc.VectorSubcoreMesh(core_axis_name="c", subcore_axis_name="s")

  @functools.partial(
      pl.kernel,
      mesh=mesh,
      compiler_params=pltpu.CompilerParams(use_tc_tiling_on_sc=True,
                                           needs_layout_passes=False),
      out_type=(
          jax.ShapeDtypeStruct((TEXT_DIM, B), jnp.float32),
          jax.ShapeDtypeStruct((TEXT_DIM, B), jnp.float32),
          jax.ShapeDtypeStruct((N_CAT, CAT_DIM, B), jnp.float32),
      ),
      scratch_types=[
          pltpu.VMEM((TEXT_VOCAB,), jnp.float32),
          pltpu.VMEM((3 * B,), jnp.int32),
          pltpu.VMEM((3 * B,), jnp.int32),
          pltpu.VMEM((B,), jnp.float32),
          pltpu.SemaphoreType.DMA,
          pltpu.SemaphoreType.DMA,
          pltpu.SemaphoreType.DMA,
          pltpu.SemaphoreType.DMA,
      ],
  )
  def k(et_hbm, ed_hbm, ct_hbm, ti_hbm, di_hbm, ci_hbm,
        t1_hbm, t2_hbm, co_hbm,
        row_v, idx0_v, idx1_v, acc_v, sem_row, sem0, sem1, sem2):
    wid = lax.axis_index("s") * NC + lax.axis_index("c")
    UNROLL = 8

    def gather_block(ibuf, nrows, first):
      UV = 4

      def vbody(v4, _):
        offs = [v4 * (UV * LANES) + u * LANES for u in range(UV)]
        gs = []
        for o in offs:
          idxs = [ibuf[pl.ds(r * B + o, LANES)] for r in range(nrows)]
          gs.append([plsc.load_gather(row_v, [ix]) for ix in idxs])
        for o, gr in zip(offs, gs):
          s = gr[0]
          for r in range(1, nrows):
            s = s + gr[r]
          if first:
            acc_v[pl.ds(o, LANES)] = s
          else:
            acc_v[pl.ds(o, LANES)] = acc_v[pl.ds(o, LANES)] + s
        return 0

      lax.fori_loop(0, NV // UV, vbody, 0)

    NBLK = L // 3

    def fire_block(idx_hbm, bi, buf, sem, n=3):
      for j in range(n):
        pltpu.async_copy(idx_hbm.at[3 * bi + j], buf.at[pl.ds(j * B, B)], sem)

    def wait_block(buf, sem, n=3):
      for j in range(n):
        pltpu.make_async_copy(ti_hbm.at[0], buf.at[pl.ds(j * B, B)],
                              sem).wait()

    def text_column(tab_hbm, idx_hbm, out_hbm, col):
      rcp = pltpu.async_copy(tab_hbm.at[col], row_v, sem_row)
      fire_block(idx_hbm, 0, idx0_v, sem0)
      fire_block(idx_hbm, 1, idx1_v, sem1)
      rcp.wait()
      wait_block(idx0_v, sem0)
      gather_block(idx0_v, 3, True)

      def pair_body(kp, _):
        fire_block(idx_hbm, 2 * kp + 2, idx0_v, sem0)
        wait_block(idx1_v, sem1)
        gather_block(idx1_v, 3, False)
        fire_block(idx_hbm, 2 * kp + 3, idx1_v, sem1)
        wait_block(idx0_v, sem0)
        gather_block(idx0_v, 3, False)
        return 0

      lax.fori_loop(0, (NBLK - 2) // 2, pair_body, 0)

      for j in range(3 * (NBLK - 1) + 3, L):
        pltpu.async_copy(idx_hbm.at[j], idx0_v.at[pl.ds((j - 48) * B, B)],
                         sem0)
      wait_block(idx1_v, sem1)
      gather_block(idx1_v, 3, False)
      wait_block(idx0_v, sem0, n=L - 48)
      gather_block(idx0_v, L - 48, False)

      scale = jnp.float32(1.0 / L)

      def scale_body(v8, _):
        base = v8 * (UNROLL * LANES)
        for u in range(UNROLL):
          off = base + u * LANES
          acc_v[pl.ds(off, LANES)] = acc_v[pl.ds(off, LANES)] * scale
        return 0
      lax.fori_loop(0, NV // UNROLL, scale_body, 0)
      pltpu.sync_copy(acc_v, out_hbm.at[col])

    for cc in range(COLS_PER_W):
      text_column(et_hbm, ti_hbm, t1_hbm, wid * COLS_PER_W + cc)
    for cc in range(COLS_PER_W):
      text_column(ed_hbm, di_hbm, t2_hbm, wid * COLS_PER_W + cc)

    def cat_gather(ibuf):
      def vbody(v8, _):
        base = v8 * (UNROLL * LANES)
        offs = [base + u * LANES for u in range(UNROLL)]
        idxs = [ibuf[pl.ds(o, LANES)] for o in offs]
        gs = [plsc.load_gather(row_v, [ix]) for ix in idxs]
        for o, g in zip(offs, gs):
          acc_v[pl.ds(o, LANES)] = g
        return 0
      lax.fori_loop(0, NV // UNROLL, vbody, 0)

    pltpu.async_copy(ci_hbm.at[0], idx0_v.at[pl.ds(0, B)], sem0)
    pltpu.async_copy(ct_hbm.at[0, wid], row_v, sem_row)

    def cat_pair(tp, _):
      for par in range(2):
        t = 2 * tp + par
        ibuf, isem = (idx0_v, sem0) if par == 0 else (idx1_v, sem1)
        nbuf, nsem = (idx1_v, sem1) if par == 0 else (idx0_v, sem0)
        pltpu.make_async_copy(ct_hbm.at[0, 0], row_v, sem_row).wait()
        pltpu.make_async_copy(ci_hbm.at[0], ibuf.at[pl.ds(0, B)], isem).wait()

        @pl.when(t + 1 < N_CAT)
        def _():
          pltpu.async_copy(ci_hbm.at[t + 1], nbuf.at[pl.ds(0, B)], nsem)

        cat_gather(ibuf)

        @pl.when(t + 1 < N_CAT)
        def _():
          pltpu.async_copy(ct_hbm.at[t + 1, wid], row_v, sem_row)

        pltpu.sync_copy(acc_v, co_hbm.at[t, wid])
      return 0

    lax.fori_loop(0, N_CAT // 2, cat_pair, 0)

  return k(emb_title_t, emb_desc_t, cat_t, title_idx_t, desc_idx_t, cat_idx_t)


def _tc_fuse(t1_t, t2_t, cat_t2d, xnum_t, num_W, num_b, W1, b1, W2, b2):
  BB = 256
  grid = (B // BB,)
  cdim = (((0,), (0,)), ((), ()))

  def body(t1_r, t2_r, cat_r, xn_r, nw_r, nb_r, w1_r, b1_r, w2_r, b2_r,
           out_r):
    f32 = jnp.float32
    num_out = jnp.maximum(
        lax.dot_general(xn_r[...], nw_r[...], cdim, preferred_element_type=f32)
        + nb_r[...], 0.0)
    h = (lax.dot_general(t1_r[...], w1_r[0:TEXT_DIM, :], cdim,
                         preferred_element_type=f32)
         + lax.dot_general(t2_r[...], w1_r[TEXT_DIM:2 * TEXT_DIM, :], cdim,
                           preferred_element_type=f32)
         + lax.dot_general(cat_r[...],
                           w1_r[2 * TEXT_DIM:2 * TEXT_DIM + N_CAT * CAT_DIM, :],
                           cdim, preferred_element_type=f32)
         + jnp.dot(num_out, w1_r[2 * TEXT_DIM + N_CAT * CAT_DIM:, :],
                   preferred_element_type=f32)
         + b1_r[...])
    h = jnp.maximum(h, 0.0)
    out_r[...] = jnp.dot(h, w2_r[...], preferred_element_type=f32) + b2_r[...]

  fusion_dim = 2 * TEXT_DIM + N_CAT * CAT_DIM + NUM_HID
  return pl.pallas_call(
      body,
      grid=grid,
      in_specs=[
          pl.BlockSpec((TEXT_DIM, BB), lambda i: (0, i)),
          pl.BlockSpec((TEXT_DIM, BB), lambda i: (0, i)),
          pl.BlockSpec((N_CAT * CAT_DIM, BB), lambda i: (0, i)),
          pl.BlockSpec((N_NUM, BB), lambda i: (0, i)),
          pl.BlockSpec((N_NUM, NUM_HID), lambda i: (0, 0)),
          pl.BlockSpec((1, NUM_HID), lambda i: (0, 0)),
          pl.BlockSpec((fusion_dim, HIDDEN), lambda i: (0, 0)),
          pl.BlockSpec((1, HIDDEN), lambda i: (0, 0)),
          pl.BlockSpec((HIDDEN, NUM_CLASSES), lambda i: (0, 0)),
          pl.BlockSpec((1, NUM_CLASSES), lambda i: (0, 0)),
      ],
      out_specs=pl.BlockSpec((BB, NUM_CLASSES), lambda i: (i, 0)),
      out_shape=jax.ShapeDtypeStruct((B, NUM_CLASSES), jnp.float32),
  )(t1_t, t2_t, cat_t2d, xnum_t, num_W, num_b, W1, b1, W2, b2)


@jax.jit
def kernel(text_title, text_description, categorical_inputs, numerical_inputs,
           emb_title, emb_desc, cat_tables, num_W, num_b, W1, b1, W2, b2):
  i32 = jnp.int32
  t1_t, t2_t, cat_out_t = _sc_features(
      emb_title.T, emb_desc.T, jnp.transpose(cat_tables, (0, 2, 1)),
      text_title.astype(i32).T, text_description.astype(i32).T,
      categorical_inputs.astype(i32).T)
  cat_t2d = cat_out_t.reshape(N_CAT * CAT_DIM, B)
  return _tc_fuse(t1_t, t2_t, cat_t2d, numerical_inputs.T, num_W,
                  num_b.reshape(1, NUM_HID), W1, b1.reshape(1, HIDDEN), W2,
                  b2.reshape(1, NUM_CLASSES))


# speedup vs baseline: 1.0146x; 1.0146x over previous
import functools

import jax
import jax.numpy as jnp
from jax import lax
from jax.experimental import pallas as pl
from jax.experimental.pallas import tpu as pltpu
from jax.experimental.pallas import tpu_sc as plsc

B = 4096
L = 50
TEXT_DIM = 64
N_CAT = 26
CAT_VOCAB = 100000
TEXT_VOCAB = 100000
CAT_DIM = 32
N_NUM = 13
NUM_HID = 64
HIDDEN = 256
NUM_CLASSES = 10

NC = 2
NS = 16
NW = NC * NS
COLS_PER_W = TEXT_DIM // NW
LANES = 16
NV = B // LANES


def _sc_features(emb_title_t, emb_desc_t, cat_t, title_idx_t, desc_idx_t,
                 cat_idx_t):
  mesh = plsc.VectorSubcoreMesh(core_axis_name="c", subcore_axis_name="s")

  @functools.partial(
      pl.kernel,
      mesh=mesh,
      compiler_params=pltpu.CompilerParams(use_tc_tiling_on_sc=True,
                                           needs_layout_passes=False),
      out_type=(
          jax.ShapeDtypeStruct((TEXT_DIM, B), jnp.float32),
          jax.ShapeDtypeStruct((TEXT_DIM, B), jnp.float32),
          jax.ShapeDtypeStruct((N_CAT, CAT_DIM, B), jnp.float32),
      ),
      scratch_types=[
          pltpu.VMEM((TEXT_VOCAB,), jnp.float32),
          pltpu.VMEM((3 * B,), jnp.int32),
          pltpu.VMEM((3 * B,), jnp.int32),
          pltpu.VMEM((B,), jnp.float32),
          pltpu.SemaphoreType.DMA,
          pltpu.SemaphoreType.DMA,
          pltpu.SemaphoreType.DMA,
          pltpu.SemaphoreType.DMA,
      ],
  )
  def k(et_hbm, ed_hbm, ct_hbm, ti_hbm, di_hbm, ci_hbm,
        t1_hbm, t2_hbm, co_hbm,
        row_v, idx0_v, idx1_v, acc_v, sem_row, sem0, sem1, sem2):
    wid = lax.axis_index("s") * NC + lax.axis_index("c")
    UNROLL = 8

    def gather_block(ibuf, nrows, first):
      UV = 4

      def vbody(v4, _):
        offs = [v4 * (UV * LANES) + u * LANES for u in range(UV)]
        gs = []
        for o in offs:
          idxs = [ibuf[pl.ds(r * B + o, LANES)] for r in range(nrows)]
          gs.append([plsc.load_gather(row_v, [ix]) for ix in idxs])
        for o, gr in zip(offs, gs):
          s = gr[0]
          for r in range(1, nrows):
            s = s + gr[r]
          if first:
            acc_v[pl.ds(o, LANES)] = s
          else:
            acc_v[pl.ds(o, LANES)] = acc_v[pl.ds(o, LANES)] + s
        return 0

      lax.fori_loop(0, NV // UV, vbody, 0)

    NBLK = L // 3

    def fire_block(idx_hbm, bi, buf, sem, n=3):
      for j in range(n):
        pltpu.async_copy(idx_hbm.at[3 * bi + j], buf.at[pl.ds(j * B, B)], sem)

    def wait_block(buf, sem, n=3):
      for j in range(n):
        pltpu.make_async_copy(ti_hbm.at[0], buf.at[pl.ds(j * B, B)],
                              sem).wait()

    def text_column(tab_hbm, idx_hbm, out_hbm, col):
      rcp = pltpu.async_copy(tab_hbm.at[col], row_v, sem_row)
      fire_block(idx_hbm, 0, idx0_v, sem0)
      fire_block(idx_hbm, 1, idx1_v, sem1)
      rcp.wait()
      wait_block(idx0_v, sem0)
      gather_block(idx0_v, 3, True)

      def pair_body(kp, _):
        fire_block(idx_hbm, 2 * kp + 2, idx0_v, sem0)
        wait_block(idx1_v, sem1)
        gather_block(idx1_v, 3, False)
        fire_block(idx_hbm, 2 * kp + 3, idx1_v, sem1)
        wait_block(idx0_v, sem0)
        gather_block(idx0_v, 3, False)
        return 0

      lax.fori_loop(0, (NBLK - 2) // 2, pair_body, 0)

      for j in range(3 * (NBLK - 1) + 3, L):
        pltpu.async_copy(idx_hbm.at[j], idx0_v.at[pl.ds((j - 48) * B, B)],
                         sem0)
      wait_block(idx1_v, sem1)
      gather_block(idx1_v, 3, False)
      wait_block(idx0_v, sem0, n=L - 48)
      gather_block(idx0_v, L - 48, False)

      pltpu.sync_copy(acc_v, out_hbm.at[col])

    for cc in range(COLS_PER_W):
      text_column(et_hbm, ti_hbm, t1_hbm, wid * COLS_PER_W + cc)
    for cc in range(COLS_PER_W):
      text_column(ed_hbm, di_hbm, t2_hbm, wid * COLS_PER_W + cc)

    def cat_gather(ibuf):
      def vbody(v8, _):
        base = v8 * (UNROLL * LANES)
        offs = [base + u * LANES for u in range(UNROLL)]
        idxs = [ibuf[pl.ds(o, LANES)] for o in offs]
        gs = [plsc.load_gather(row_v, [ix]) for ix in idxs]
        for o, g in zip(offs, gs):
          acc_v[pl.ds(o, LANES)] = g
        return 0
      lax.fori_loop(0, NV // UNROLL, vbody, 0)

    pltpu.async_copy(ci_hbm.at[0], idx0_v.at[pl.ds(0, B)], sem0)
    pltpu.async_copy(ct_hbm.at[0, wid], row_v, sem_row)

    def cat_pair(tp, _):
      for par in range(2):
        t = 2 * tp + par
        ibuf, isem = (idx0_v, sem0) if par == 0 else (idx1_v, sem1)
        nbuf, nsem = (idx1_v, sem1) if par == 0 else (idx0_v, sem0)
        pltpu.make_async_copy(ct_hbm.at[0, 0], row_v, sem_row).wait()
        pltpu.make_async_copy(ci_hbm.at[0], ibuf.at[pl.ds(0, B)], isem).wait()

        @pl.when(t + 1 < N_CAT)
        def _():
          pltpu.async_copy(ci_hbm.at[t + 1], nbuf.at[pl.ds(0, B)], nsem)

        cat_gather(ibuf)

        @pl.when(t + 1 < N_CAT)
        def _():
          pltpu.async_copy(ct_hbm.at[t + 1, wid], row_v, sem_row)

        pltpu.sync_copy(acc_v, co_hbm.at[t, wid])
      return 0

    lax.fori_loop(0, N_CAT // 2, cat_pair, 0)

  return k(emb_title_t, emb_desc_t, cat_t, title_idx_t, desc_idx_t, cat_idx_t)


def _tc_fuse(t1_t, t2_t, cat_t2d, xnum_t, num_W, num_b, W1, b1, W2, b2):
  BB = 256
  grid = (B // BB,)
  cdim = (((0,), (0,)), ((), ()))

  inv_l = 1.0 / L

  def body(t1_r, t2_r, cat_r, xn_r, nw_r, nb_r, w1_r, b1_r, w2_r, b2_r,
           out_r):
    f32 = jnp.float32
    num_out = jnp.maximum(
        lax.dot_general(xn_r[...], nw_r[...], cdim, preferred_element_type=f32)
        + nb_r[...], 0.0)
    h = (lax.dot_general(t1_r[...], w1_r[0:TEXT_DIM, :] * inv_l, cdim,
                         preferred_element_type=f32)
         + lax.dot_general(t2_r[...], w1_r[TEXT_DIM:2 * TEXT_DIM, :] * inv_l,
                           cdim, preferred_element_type=f32)
         + lax.dot_general(cat_r[...],
                           w1_r[2 * TEXT_DIM:2 * TEXT_DIM + N_CAT * CAT_DIM, :],
                           cdim, preferred_element_type=f32)
         + jnp.dot(num_out, w1_r[2 * TEXT_DIM + N_CAT * CAT_DIM:, :],
                   preferred_element_type=f32)
         + b1_r[...])
    h = jnp.maximum(h, 0.0)
    out_r[...] = (lax.dot_general(w2_r[...], h, (((0,), (1,)), ((), ())),
                                  preferred_element_type=f32)
                  + b2_r[...].reshape(NUM_CLASSES, 1))

  fusion_dim = 2 * TEXT_DIM + N_CAT * CAT_DIM + NUM_HID
  return pl.pallas_call(
      body,
      grid=grid,
      in_specs=[
          pl.BlockSpec((TEXT_DIM, BB), lambda i: (0, i)),
          pl.BlockSpec((TEXT_DIM, BB), lambda i: (0, i)),
          pl.BlockSpec((N_CAT * CAT_DIM, BB), lambda i: (0, i)),
          pl.BlockSpec((N_NUM, BB), lambda i: (0, i)),
          pl.BlockSpec((N_NUM, NUM_HID), lambda i: (0, 0)),
          pl.BlockSpec((1, NUM_HID), lambda i: (0, 0)),
          pl.BlockSpec((fusion_dim, HIDDEN), lambda i: (0, 0)),
          pl.BlockSpec((1, HIDDEN), lambda i: (0, 0)),
          pl.BlockSpec((HIDDEN, NUM_CLASSES), lambda i: (0, 0)),
          pl.BlockSpec((1, NUM_CLASSES), lambda i: (0, 0)),
      ],
      out_specs=pl.BlockSpec((NUM_CLASSES, BB), lambda i: (0, i)),
      out_shape=jax.ShapeDtypeStruct((NUM_CLASSES, B), jnp.float32),
  )(t1_t, t2_t, cat_t2d, xnum_t, num_W, num_b, W1, b1, W2, b2)


@jax.jit
def kernel(text_title, text_description, categorical_inputs, numerical_inputs,
           emb_title, emb_desc, cat_tables, num_W, num_b, W1, b1, W2, b2):
  i32 = jnp.int32
  t1_t, t2_t, cat_out_t = _sc_features(
      emb_title.T, emb_desc.T, jnp.transpose(cat_tables, (0, 2, 1)),
      text_title.astype(i32).T, text_description.astype(i32).T,
      categorical_inputs.astype(i32).T)
  cat_t2d = cat_out_t.reshape(N_CAT * CAT_DIM, B)
  logits_t = _tc_fuse(t1_t, t2_t, cat_t2d, numerical_inputs.T, num_W,
                      num_b.reshape(1, NUM_HID), W1, b1.reshape(1, HIDDEN),
                      W2, b2.reshape(1, NUM_CLASSES))
  return logits_t.T
